# Initial kernel scaffold; baseline (speedup 1.0000x reference)
import functools

import jax
import jax.numpy as jnp
from jax import lax
from jax.experimental import pallas as pl
from jax.experimental.pallas import tpu as pltpu
from jax.experimental.pallas import tpu_sc as plsc

R = 128
N = 32768
K = 64
NC = 2
NS = 16
L = 16
NW = NC * NS
RPW = R // NW
GV = 8
GE = GV * L
NG = N // GE
NGV = NG // L

_BIG = jnp.int32(2**31 - 1)


def _row_topk(xv, m2, outv, outi):
    lanes = lax.iota(jnp.int32, L)
    lane0 = lanes == 0
    neg = jnp.float32(-jnp.inf)

    def build_group(g, carry):
        acc = xv[pl.ds(g * GE, L)]
        for j in range(1, GV):
            acc = jnp.maximum(acc, xv[pl.ds(g * GE + j * L, L)])
        gm = jnp.max(acc)
        plsc.store_scatter(
            m2, [jnp.full((L,), g, jnp.int32)],
            jnp.full((L,), gm, jnp.float32), mask=lane0)
        return carry

    lax.fori_loop(0, NG, build_group, 0)

    def extract(k, carry):
        acc = m2[pl.ds(0, L)]
        for c in range(1, NGV):
            acc = jnp.maximum(acc, m2[pl.ds(c * L, L)])
        gmax = jnp.max(acc)
        gacc = jnp.full((L,), _BIG, jnp.int32)
        for c in range(NGV):
            v = m2[pl.ds(c * L, L)]
            gacc = jnp.minimum(gacc, jnp.where(v == gmax, lanes + (c * L), _BIG))
        gstar = jnp.min(gacc)
        base = gstar * GE
        iacc = jnp.full((L,), _BIG, jnp.int32)
        for j in range(GV):
            v = xv[pl.ds(base + j * L, L)]
            iacc = jnp.minimum(iacc, jnp.where(v == gmax, lanes + (base + j * L), _BIG))
        bi = jnp.min(iacc)
        kidx = jnp.full((L,), k, jnp.int32)
        plsc.store_scatter(outv, [kidx], jnp.full((L,), gmax, jnp.float32),
                           mask=lane0)
        plsc.store_scatter(outi, [kidx], jnp.full((L,), bi, jnp.int32),
                           mask=lane0)
        q = (bi >> 4) << 4
        lane = bi - q
        vq = xv[pl.ds(q, L)]
        xv[pl.ds(q, L)] = jnp.where(lanes == lane, neg, vq)
        acc2 = xv[pl.ds(base, L)]
        for j in range(1, GV):
            acc2 = jnp.maximum(acc2, xv[pl.ds(base + j * L, L)])
        plsc.store_scatter(
            m2, [jnp.full((L,), gstar, jnp.int32)],
            jnp.full((L,), jnp.max(acc2), jnp.float32), mask=lane0)
        return carry

    lax.fori_loop(0, K, extract, 0)


@functools.partial(
    pl.kernel,
    out_type=(
        jax.ShapeDtypeStruct((R, K), jnp.float32),
        jax.ShapeDtypeStruct((R, K), jnp.int32),
    ),
    mesh=plsc.VectorSubcoreMesh(
        core_axis_name="c", subcore_axis_name="s",
        num_cores=NC, num_subcores=NS),
    scratch_types=[
        pltpu.VMEM((N,), jnp.float32),
        pltpu.VMEM((NG,), jnp.float32),
        pltpu.VMEM((K,), jnp.float32),
        pltpu.VMEM((K,), jnp.int32),
    ],
)
def _topk_sc(x_hbm, vals_hbm, idx_hbm, xv, m2, outv, outi):
    wid = lax.axis_index("s") * NC + lax.axis_index("c")

    def row_body(r, carry):
        row = wid * RPW + r
        pltpu.sync_copy(x_hbm.at[row], xv)
        _row_topk(xv, m2, outv, outi)
        pltpu.sync_copy(outv, vals_hbm.at[row])
        pltpu.sync_copy(outi, idx_hbm.at[row])
        return carry

    lax.fori_loop(0, RPW, row_body, 0)


def kernel(x):
    vals, idx = _topk_sc(x)
    return vals, idx.astype(jnp.int64)


# speedup vs baseline: 8.8197x; 8.8197x over previous
import functools

import jax
import jax.numpy as jnp
from jax import lax
from jax.experimental import pallas as pl
from jax.experimental.pallas import tpu as pltpu
from jax.experimental.pallas import tpu_sc as plsc

R = 128
N = 32768
K = 64
NC = 2
NS = 16
L = 16
NW = NC * NS
RPW = R // NW
GV = 8
GE = GV * L
NG = N // GE
NGV = NG // L

_BIG = 2**31 - 1


def _row_topk(xv, m2, outv, outi):
    lanes = lax.iota(jnp.int32, L)
    lane0 = lanes == 0
    neg = jnp.float32(-jnp.inf)

    def build_group(g, carry):
        acc = xv[pl.ds(g * GE, L)]
        for j in range(1, GV):
            acc = jnp.maximum(acc, xv[pl.ds(g * GE + j * L, L)])
        gm = jnp.max(acc)
        plsc.store_scatter(
            m2, [jnp.full((L,), g, jnp.int32)],
            jnp.full((L,), gm, jnp.float32), mask=lane0)
        return carry

    lax.fori_loop(0, NG, build_group, 0)

    def extract(k, carry):
        acc = m2[pl.ds(0, L)]
        for c in range(1, NGV):
            acc = jnp.maximum(acc, m2[pl.ds(c * L, L)])
        gmax = jnp.max(acc)
        gacc = jnp.full((L,), _BIG, jnp.int32)
        for c in range(NGV):
            v = m2[pl.ds(c * L, L)]
            gacc = jnp.minimum(gacc, jnp.where(v == gmax, lanes + (c * L), _BIG))
        gstar = jnp.min(gacc)
        base = gstar * GE
        iacc = jnp.full((L,), _BIG, jnp.int32)
        for j in range(GV):
            v = xv[pl.ds(base + j * L, L)]
            iacc = jnp.minimum(iacc, jnp.where(v == gmax, lanes + (base + j * L), _BIG))
        bi = jnp.min(iacc)
        kidx = jnp.full((L,), k, jnp.int32)
        plsc.store_scatter(outv, [kidx], jnp.full((L,), gmax, jnp.float32),
                           mask=lane0)
        plsc.store_scatter(outi, [kidx], jnp.full((L,), bi, jnp.int32),
                           mask=lane0)
        q = (bi >> 4) << 4
        lane = bi - q
        vq = xv[pl.ds(q, L)]
        xv[pl.ds(q, L)] = jnp.where(lanes == lane, neg, vq)
        acc2 = xv[pl.ds(base, L)]
        for j in range(1, GV):
            acc2 = jnp.maximum(acc2, xv[pl.ds(base + j * L, L)])
        plsc.store_scatter(
            m2, [jnp.full((L,), gstar, jnp.int32)],
            jnp.full((L,), jnp.max(acc2), jnp.float32), mask=lane0)
        return carry

    lax.fori_loop(0, K, extract, 0)


@functools.partial(
    pl.kernel,
    out_type=(
        jax.ShapeDtypeStruct((R, K), jnp.float32),
        jax.ShapeDtypeStruct((R, K), jnp.int32),
    ),
    mesh=plsc.VectorSubcoreMesh(
        core_axis_name="c", subcore_axis_name="s",
        num_cores=NC, num_subcores=NS),
    compiler_params=pltpu.CompilerParams(needs_layout_passes=False),
    scratch_types=[
        pltpu.VMEM((N,), jnp.float32),
        pltpu.VMEM((NG,), jnp.float32),
        pltpu.VMEM((K,), jnp.float32),
        pltpu.VMEM((K,), jnp.int32),
    ],
)
def _topk_sc(x_hbm, vals_hbm, idx_hbm, xv, m2, outv, outi):
    wid = lax.axis_index("s") * NC + lax.axis_index("c")

    def row_body(r, carry):
        row = wid * RPW + r
        pltpu.sync_copy(x_hbm.at[row], xv)
        _row_topk(xv, m2, outv, outi)
        pltpu.sync_copy(outv, vals_hbm.at[row])
        pltpu.sync_copy(outi, idx_hbm.at[row])
        return carry

    lax.fori_loop(0, RPW, row_body, 0)


def kernel(x):
    vals, idx = _topk_sc(x)
    return vals, idx.astype(jnp.int64)
